# Initial kernel scaffold; baseline (speedup 1.0000x reference)
import jax
import jax.numpy as jnp
from jax.experimental import pallas as pl


def kernel(input, weight):
    raise NotImplementedError("write your pallas kernel here")


# speedup vs baseline: 1.1016x; 1.1016x over previous
import functools

import jax
import jax.numpy as jnp
from jax import lax
from jax.experimental import pallas as pl
from jax.experimental.pallas import tpu as pltpu
from jax.experimental.pallas import tpu_sc as plsc

EMBED_D = 32
CHUNK = 1024


@functools.lru_cache(maxsize=None)
def _make_gather(B, V):
    info = plsc.get_sparse_core_info()
    NC, NS = info.num_cores, info.num_subcores
    NW = NC * NS
    assert B % (NW * CHUNK) == 0
    b_per_w = B // NW
    n_chunks = b_per_w // CHUNK
    mesh = plsc.VectorSubcoreMesh(core_axis_name="c", subcore_axis_name="s")

    @functools.partial(
        pl.kernel,
        mesh=mesh,
        compiler_params=pltpu.CompilerParams(use_tc_tiling_on_sc=False),
        out_type=jax.ShapeDtypeStruct((B, EMBED_D), jnp.float32),
        scratch_types=[
            pltpu.VMEM((CHUNK,), jnp.int32),
            pltpu.VMEM((CHUNK, EMBED_D), jnp.float32),
            pltpu.SemaphoreType.DMA,
        ],
    )
    def gather_kernel(idx_hbm, table_hbm, out_hbm, idx_v, rows_v, sem):
        wid = lax.axis_index("s") * NC + lax.axis_index("c")
        base = wid * b_per_w

        def body(i, carry):
            off = base + i * CHUNK
            pltpu.sync_copy(idx_hbm.at[pl.ds(off, CHUNK)], idx_v)
            pltpu.async_copy(table_hbm.at[idx_v], rows_v, sem).wait()
            pltpu.sync_copy(rows_v, out_hbm.at[pl.ds(off, CHUNK)])
            return carry

        lax.fori_loop(0, n_chunks, body, 0)

    return gather_kernel


def kernel(input, weight):
    idx = jnp.reshape(input, (-1,)).astype(jnp.int32)
    B = idx.shape[0]
    out = _make_gather(B, weight.shape[0])(idx, weight)
    return jnp.reshape(out, (*input.shape, EMBED_D))


# speedup vs baseline: 1.1109x; 1.0084x over previous
import functools

import jax
import jax.numpy as jnp
from jax import lax
from jax.experimental import pallas as pl
from jax.experimental.pallas import tpu as pltpu
from jax.experimental.pallas import tpu_sc as plsc

EMBED_D = 32
CHUNK = 1024


@functools.lru_cache(maxsize=None)
def _make_gather(B, V):
    info = plsc.get_sparse_core_info()
    NC, NS = info.num_cores, info.num_subcores
    NW = NC * NS
    assert B % (NW * 2 * CHUNK) == 0
    b_per_w = B // NW
    n_chunks = b_per_w // CHUNK
    n_pairs = n_chunks // 2
    mesh = plsc.VectorSubcoreMesh(core_axis_name="c", subcore_axis_name="s")

    @functools.partial(
        pl.kernel,
        mesh=mesh,
        compiler_params=pltpu.CompilerParams(use_tc_tiling_on_sc=False),
        out_type=jax.ShapeDtypeStruct((B, EMBED_D), jnp.float32),
        scratch_types=[
            pltpu.VMEM((b_per_w,), jnp.int32),
            pltpu.VMEM((CHUNK, EMBED_D), jnp.float32),
            pltpu.VMEM((CHUNK, EMBED_D), jnp.float32),
            pltpu.SemaphoreType.DMA,
            pltpu.SemaphoreType.DMA,
            pltpu.SemaphoreType.DMA,
            pltpu.SemaphoreType.DMA,
        ],
    )
    def gather_kernel(idx_hbm, table_hbm, out_hbm, idx_v, rows0, rows1,
                      sem_g0, sem_g1, sem_o0, sem_o1):
        wid = lax.axis_index("s") * NC + lax.axis_index("c")
        base = wid * b_per_w

        pltpu.sync_copy(idx_hbm.at[pl.ds(base, b_per_w)], idx_v)

        def gather(chunk, rows, sem):
            pltpu.async_copy(
                table_hbm.at[idx_v.at[pl.ds(chunk * CHUNK, CHUNK)]], rows, sem)

        def writeback(rows, chunk, sem):
            pltpu.async_copy(
                rows, out_hbm.at[pl.ds(base + chunk * CHUNK, CHUNK)], sem)

        def wait_gather(rows, sem):
            pltpu.make_async_copy(
                table_hbm.at[idx_v.at[pl.ds(0, CHUNK)]], rows, sem).wait()

        def wait_out(rows, sem):
            pltpu.make_async_copy(
                rows, out_hbm.at[pl.ds(base, CHUNK)], sem).wait()

        gather(0, rows0, sem_g0)

        def body(j, carry):
            i0 = 2 * j
            wait_gather(rows0, sem_g0)

            @pl.when(j > 0)
            def _():
                wait_out(rows1, sem_o1)

            gather(i0 + 1, rows1, sem_g1)
            writeback(rows0, i0, sem_o0)
            wait_gather(rows1, sem_g1)
            wait_out(rows0, sem_o0)
            nxt = lax.rem(i0 + 2, n_chunks)
            gather(nxt, rows0, sem_g0)
            writeback(rows1, i0 + 1, sem_o1)
            return carry

        lax.fori_loop(0, n_pairs, body, 0)
        wait_gather(rows0, sem_g0)
        wait_out(rows1, sem_o1)

    return gather_kernel


def kernel(input, weight):
    idx = jnp.reshape(input, (-1,)).astype(jnp.int32)
    B = idx.shape[0]
    out = _make_gather(B, weight.shape[0])(idx, weight)
    return jnp.reshape(out, (*input.shape, EMBED_D))


# speedup vs baseline: 3.1160x; 2.8049x over previous
import functools

import jax
import jax.numpy as jnp
from jax import lax
from jax.experimental import pallas as pl
from jax.experimental.pallas import tpu as pltpu
from jax.experimental.pallas import tpu_sc as plsc

D = 32
S = 100
CB = 16


@functools.lru_cache(maxsize=None)
def _make_gather(BT, V):
    info = plsc.get_sparse_core_info()
    NC, NS, L = info.num_cores, info.num_subcores, info.num_lanes
    NW = NC * NS
    assert L == CB
    assert BT % (NW * CB) == 0
    b_per_w = BT // NW
    n_chunks = b_per_w // CB
    mesh = plsc.VectorSubcoreMesh(core_axis_name="c", subcore_axis_name="s")

    @functools.partial(
        pl.kernel,
        mesh=mesh,
        compiler_params=pltpu.CompilerParams(
            use_tc_tiling_on_sc=False, needs_layout_passes=False),
        out_type=jax.ShapeDtypeStruct((S, D, BT), jnp.float32),
        scratch_types=[
            pltpu.VMEM((S, CB), jnp.int32),
            pltpu.VMEM((S, CB, D), jnp.float32),
            pltpu.VMEM((S, D, CB), jnp.float32),
            pltpu.SemaphoreType.DMA,
        ],
    )
    def gather_kernel(idx_hbm, table_hbm, out_hbm, idxb, rows, outb, sem):
        wid = lax.axis_index("s") * NC + lax.axis_index("c")
        b_base = wid * b_per_w
        iota = lax.iota(jnp.int32, L)

        def chunk_body(c, carry):
            b0 = b_base + c * CB
            pltpu.sync_copy(idx_hbm.at[:, pl.ds(b0, CB)], idxb)

            def issue(s, carry2):
                pltpu.async_copy(table_hbm.at[idxb.at[s]], rows.at[s], sem)
                return carry2

            lax.fori_loop(0, S, issue, 0)

            def drain(s, carry2):
                pltpu.make_async_copy(
                    table_hbm.at[idxb.at[0]], rows.at[0], sem).wait()
                return carry2

            lax.fori_loop(0, S, drain, 0)

            def s_body(s, carry2):
                s_vec = jnp.full((L,), 0, jnp.int32) + s
                for d in range(D):
                    v = plsc.load_gather(
                        rows, [s_vec, iota, jnp.full((L,), d, jnp.int32)])
                    outb[s, d, :] = v
                return carry2

            lax.fori_loop(0, S, s_body, 0)
            pltpu.sync_copy(outb, out_hbm.at[:, :, pl.ds(b0, CB)])
            return carry

        lax.fori_loop(0, n_chunks, chunk_body, 0)

    return gather_kernel


def kernel(input, weight):
    idx_t = jnp.transpose(input).astype(jnp.int32)
    out_t = _make_gather(idx_t.shape[1], weight.shape[0])(idx_t, weight)
    return jnp.transpose(out_t, (2, 0, 1))


# speedup vs baseline: 3.3774x; 1.0839x over previous
import functools

import jax
import jax.numpy as jnp
from jax import lax
from jax.experimental import pallas as pl
from jax.experimental.pallas import tpu as pltpu
from jax.experimental.pallas import tpu_sc as plsc

D = 32
S = 100
SH = 50
CB = 16
RPC = SH * CB


@functools.lru_cache(maxsize=None)
def _make_gather(BT, V):
    info = plsc.get_sparse_core_info()
    NC, NS, L = info.num_cores, info.num_subcores, info.num_lanes
    NW = NC * NS
    assert L == CB and S % SH == 0
    assert BT % (NW * CB) == 0
    b_per_w = BT // NW
    n_chunks = (b_per_w // CB) * (S // SH)
    mesh = plsc.VectorSubcoreMesh(core_axis_name="c", subcore_axis_name="s")

    @functools.partial(
        pl.kernel,
        mesh=mesh,
        compiler_params=pltpu.CompilerParams(
            use_tc_tiling_on_sc=False, needs_layout_passes=False),
        out_type=jax.ShapeDtypeStruct((S, D, BT), jnp.float32),
        scratch_types=[
            pltpu.VMEM((2, SH, CB), jnp.int32),
            pltpu.VMEM((2, RPC), jnp.int32),
            pltpu.VMEM((2, RPC, D), jnp.float32),
            pltpu.VMEM((2, SH, D, CB), jnp.float32),
            pltpu.SemaphoreType.DMA,
            pltpu.SemaphoreType.DMA,
            pltpu.SemaphoreType.DMA,
            pltpu.SemaphoreType.DMA,
        ],
    )
    def gather_kernel(idx_hbm, table_hbm, out_hbm, idxb, fidx, rows, outb,
                      sem_g0, sem_g1, sem_o0, sem_o1):
        wid = lax.axis_index("s") * NC + lax.axis_index("c")
        b_base = wid * b_per_w
        iota = lax.iota(jnp.int32, L)

        def chunk_coords(i):
            s0 = lax.rem(i, 2) * SH
            b0 = b_base + lax.div(i, 2) * CB
            return s0, b0

        def stage_and_fire(i, p):
            s0, b0 = chunk_coords(i)
            pltpu.sync_copy(
                idx_hbm.at[pl.ds(s0, SH), pl.ds(b0, CB)], idxb.at[p])

            def flat_body(s, carry):
                fidx[p, pl.ds(s * CB, CB)] = idxb[p, s, :]
                return carry

            lax.fori_loop(0, SH, flat_body, 0)
            pltpu.async_copy(table_hbm.at[fidx.at[p]], rows.at[p],
                             sem_g0 if p == 0 else sem_g1)

        def wait_gather(p):
            pltpu.make_async_copy(
                table_hbm.at[pl.ds(0, RPC)], rows.at[p],
                sem_g0 if p == 0 else sem_g1).wait()

        def wait_out(p):
            s0 = 0
            pltpu.make_async_copy(
                outb.at[p],
                out_hbm.at[pl.ds(s0, SH), :, pl.ds(b_base, CB)],
                sem_o0 if p == 0 else sem_o1).wait()

        def transpose_and_fire(i, p):
            def s_body(s, carry):
                row_vec = iota + s * CB
                for d in range(D):
                    v = plsc.load_gather(
                        rows.at[p], [row_vec, jnp.full((L,), d, jnp.int32)])
                    outb[p, s, d, :] = v
                return carry

            lax.fori_loop(0, SH, s_body, 0)
            s0, b0 = chunk_coords(i)
            pltpu.async_copy(
                outb.at[p],
                out_hbm.at[pl.ds(s0, SH), :, pl.ds(b0, CB)],
                sem_o0 if p == 0 else sem_o1)

        stage_and_fire(0, 0)

        def body(i, carry):
            p = lax.rem(i, 2)

            @pl.when(i + 1 < n_chunks)
            def _():
                @pl.when(p == 0)
                def _():
                    stage_and_fire(i + 1, 1)

                @pl.when(p == 1)
                def _():
                    stage_and_fire(i + 1, 0)

            @pl.when(p == 0)
            def _():
                wait_gather(0)

                @pl.when(i >= 2)
                def _():
                    wait_out(0)

                transpose_and_fire(i, 0)

            @pl.when(p == 1)
            def _():
                wait_gather(1)

                @pl.when(i >= 2)
                def _():
                    wait_out(1)

                transpose_and_fire(i, 1)

            return carry

        lax.fori_loop(0, n_chunks, body, 0)
        wait_out(0)
        wait_out(1)

    return gather_kernel


def kernel(input, weight):
    idx_t = jnp.transpose(input).astype(jnp.int32)
    out_t = _make_gather(idx_t.shape[1], weight.shape[0])(idx_t, weight)
    return jnp.transpose(out_t, (2, 0, 1))


# speedup vs baseline: 5.2876x; 1.5656x over previous
import functools

import jax
import jax.numpy as jnp
from jax import lax
from jax.experimental import pallas as pl
from jax.experimental.pallas import tpu as pltpu
from jax.experimental.pallas import tpu_sc as plsc

D = 32
S = 100
SH = 50
CB = 16
RPC = SH * CB


@functools.lru_cache(maxsize=None)
def _make_gather(BT, V):
    info = plsc.get_sparse_core_info()
    NC, NS, L = info.num_cores, info.num_subcores, info.num_lanes
    NW = NC * NS
    assert L == CB and S % SH == 0
    assert BT % (NW * CB) == 0
    b_per_w = BT // NW
    n_chunks = (b_per_w // CB) * (S // SH)
    mesh = plsc.VectorSubcoreMesh(core_axis_name="c", subcore_axis_name="s")

    @functools.partial(
        pl.kernel,
        mesh=mesh,
        compiler_params=pltpu.CompilerParams(
            use_tc_tiling_on_sc=False, needs_layout_passes=False),
        out_type=jax.ShapeDtypeStruct((S, D, BT), jnp.float32),
        scratch_types=[
            pltpu.VMEM((2, SH, CB), jnp.int32),
            pltpu.VMEM((2, RPC), jnp.int32),
            pltpu.VMEM((2, RPC, D), jnp.float32),
            pltpu.VMEM((2, SH, D, CB), jnp.float32),
            pltpu.SemaphoreType.DMA,
            pltpu.SemaphoreType.DMA,
            pltpu.SemaphoreType.DMA,
            pltpu.SemaphoreType.DMA,
        ],
    )
    def gather_kernel(idx_hbm, table_hbm, out_hbm, idxb, fidx, rows, outb,
                      sem_g0, sem_g1, sem_o0, sem_o1):
        wid = lax.axis_index("s") * NC + lax.axis_index("c")
        b_base = wid * b_per_w
        iota = lax.iota(jnp.int32, L)

        def chunk_coords(i):
            s0 = lax.rem(i, 2) * SH
            b0 = b_base + lax.div(i, 2) * CB
            return s0, b0

        def stage_and_fire(i, p):
            s0, b0 = chunk_coords(i)
            pltpu.sync_copy(
                idx_hbm.at[pl.ds(s0, SH), pl.ds(b0, CB)], idxb.at[p])

            def flat_body(s, carry):
                fidx[p, pl.ds(s * CB, CB)] = idxb[p, s, :]
                return carry

            lax.fori_loop(0, SH, flat_body, 0)
            pltpu.async_copy(table_hbm.at[fidx.at[p]], rows.at[p],
                             sem_g0 if p == 0 else sem_g1)

        def wait_gather(p):
            pltpu.make_async_copy(
                table_hbm.at[pl.ds(0, RPC)], rows.at[p],
                sem_g0 if p == 0 else sem_g1).wait()

        def wait_out(p):
            s0 = 0
            pltpu.make_async_copy(
                outb.at[p],
                out_hbm.at[pl.ds(s0, SH), :, pl.ds(b_base, CB)],
                sem_o0 if p == 0 else sem_o1).wait()

        iota_hi = iota + CB

        def transpose_and_fire(i, p):
            def s_body(s, carry):
                s_vec = jnp.full((L,), 0, jnp.int32) + s
                for b in range(CB):
                    row = s * CB + b
                    b_vec = jnp.full((L,), b, jnp.int32)
                    v_lo = rows[p, row, pl.ds(0, L)]
                    v_hi = rows[p, row, pl.ds(L, L)]
                    plsc.store_scatter(
                        outb.at[p], [s_vec, iota, b_vec], v_lo)
                    plsc.store_scatter(
                        outb.at[p], [s_vec, iota_hi, b_vec], v_hi)
                return carry

            lax.fori_loop(0, SH, s_body, 0)
            s0, b0 = chunk_coords(i)
            pltpu.async_copy(
                outb.at[p],
                out_hbm.at[pl.ds(s0, SH), :, pl.ds(b0, CB)],
                sem_o0 if p == 0 else sem_o1)

        stage_and_fire(0, 0)

        def body(i, carry):
            p = lax.rem(i, 2)

            @pl.when(i + 1 < n_chunks)
            def _():
                @pl.when(p == 0)
                def _():
                    stage_and_fire(i + 1, 1)

                @pl.when(p == 1)
                def _():
                    stage_and_fire(i + 1, 0)

            @pl.when(p == 0)
            def _():
                wait_gather(0)

                @pl.when(i >= 2)
                def _():
                    wait_out(0)

                transpose_and_fire(i, 0)

            @pl.when(p == 1)
            def _():
                wait_gather(1)

                @pl.when(i >= 2)
                def _():
                    wait_out(1)

                transpose_and_fire(i, 1)

            return carry

        lax.fori_loop(0, n_chunks, body, 0)
        wait_out(0)
        wait_out(1)

    return gather_kernel


def kernel(input, weight):
    idx_t = jnp.transpose(input).astype(jnp.int32)
    out_t = _make_gather(idx_t.shape[1], weight.shape[0])(idx_t, weight)
    return jnp.transpose(out_t, (2, 0, 1))


# speedup vs baseline: 5.4519x; 1.0311x over previous
import functools

import jax
import jax.numpy as jnp
from jax import lax
from jax.experimental import pallas as pl
from jax.experimental.pallas import tpu as pltpu
from jax.experimental.pallas import tpu_sc as plsc

D = 32
DP = 128
S = 100
SH = 20
CB = 16
RPC = SH * CB


@functools.lru_cache(maxsize=None)
def _make_gather(BT, V):
    info = plsc.get_sparse_core_info()
    NC, NS, L = info.num_cores, info.num_subcores, info.num_lanes
    NW = NC * NS
    assert L == CB and S % SH == 0
    assert BT % (NW * CB) == 0
    b_per_w = BT // NW
    n_chunks = (b_per_w // CB) * (S // SH)
    s_blocks = S // SH
    mesh = plsc.VectorSubcoreMesh(core_axis_name="c", subcore_axis_name="s")

    @functools.partial(
        pl.kernel,
        mesh=mesh,
        compiler_params=pltpu.CompilerParams(
            use_tc_tiling_on_sc=False, needs_layout_passes=False),
        out_type=jax.ShapeDtypeStruct((S, D // 8, BT // 128, 8, 128),
                                      jnp.float32),
        scratch_types=[
            pltpu.VMEM((2, SH, CB), jnp.int32),
            pltpu.VMEM((2, RPC), jnp.int32),
            pltpu.VMEM((2, RPC, DP), jnp.float32),
            pltpu.VMEM((2, SH, D // 8, 8, CB), jnp.float32),
            pltpu.SemaphoreType.DMA,
            pltpu.SemaphoreType.DMA,
            pltpu.SemaphoreType.DMA,
            pltpu.SemaphoreType.DMA,
        ],
    )
    def gather_kernel(idx_hbm, table_hbm, out_hbm, idxb, fidx, rows, outb,
                      sem_g0, sem_g1, sem_o0, sem_o1):
        wid = lax.axis_index("s") * NC + lax.axis_index("c")
        b_base = wid * b_per_w
        iota = lax.iota(jnp.int32, L)
        ti_lo = lax.div(iota, 8)
        ti_hi = ti_lo + 2
        rl = lax.rem(iota, 8)

        def chunk_coords(i):
            s0 = lax.rem(i, s_blocks) * SH
            b0 = b_base + lax.div(i, s_blocks) * CB
            return s0, b0

        def stage_and_fire(i, p):
            s0, b0 = chunk_coords(i)
            pltpu.sync_copy(
                idx_hbm.at[pl.ds(s0, SH), pl.ds(b0, CB)], idxb.at[p])

            def flat_body(s, carry):
                fidx[p, pl.ds(s * CB, CB)] = idxb[p, s, :]
                return carry

            lax.fori_loop(0, SH, flat_body, 0)
            pltpu.async_copy(table_hbm.at[fidx.at[p]],
                             rows.at[p], sem_g0 if p == 0 else sem_g1)

        def wait_gather(p):
            pltpu.make_async_copy(
                table_hbm.at[pl.ds(0, RPC)], rows.at[p],
                sem_g0 if p == 0 else sem_g1).wait()

        def out_slice(s0, tj, cl):
            return out_hbm.at[pl.ds(s0, SH), :, tj, :, pl.ds(cl, CB)]

        def wait_out(p):
            pltpu.make_async_copy(
                outb.at[p], out_slice(0, 0, b_base % 128),
                sem_o0 if p == 0 else sem_o1).wait()

        def transpose_and_fire(i, p):
            def s_body(s, carry):
                s_vec = jnp.full((L,), 0, jnp.int32) + s
                for b in range(CB):
                    row = s * CB + b
                    b_vec = jnp.full((L,), b, jnp.int32)
                    v_lo = rows[p, row, pl.ds(0, L)]
                    v_hi = rows[p, row, pl.ds(L, L)]
                    plsc.store_scatter(
                        outb.at[p], [s_vec, ti_lo, rl, b_vec], v_lo)
                    plsc.store_scatter(
                        outb.at[p], [s_vec, ti_hi, rl, b_vec], v_hi)
                return carry

            lax.fori_loop(0, SH, s_body, 0)
            s0, b0 = chunk_coords(i)
            pltpu.async_copy(
                outb.at[p],
                out_slice(s0, lax.div(b0, 128), lax.rem(b0, 128)),
                sem_o0 if p == 0 else sem_o1)

        stage_and_fire(0, 0)

        def body(i, carry):
            p = lax.rem(i, 2)

            @pl.when(i + 1 < n_chunks)
            def _():
                @pl.when(p == 0)
                def _():
                    stage_and_fire(i + 1, 1)

                @pl.when(p == 1)
                def _():
                    stage_and_fire(i + 1, 0)

            @pl.when(p == 0)
            def _():
                wait_gather(0)

                @pl.when(i >= 2)
                def _():
                    wait_out(0)

                transpose_and_fire(i, 0)

            @pl.when(p == 1)
            def _():
                wait_gather(1)

                @pl.when(i >= 2)
                def _():
                    wait_out(1)

                transpose_and_fire(i, 1)

            return carry

        lax.fori_loop(0, n_chunks, body, 0)
        wait_out(0)
        wait_out(1)

    return gather_kernel


def kernel(input, weight):
    idx_t = jnp.transpose(input).astype(jnp.int32)
    wp = jnp.pad(weight, ((0, 0), (0, DP - D)))
    out5 = _make_gather(idx_t.shape[1], weight.shape[0])(idx_t, wp)
    BT = idx_t.shape[1]
    out = jnp.transpose(out5, (2, 4, 0, 1, 3))
    return jnp.reshape(out, (BT, S, D))


# speedup vs baseline: 5.7733x; 1.0589x over previous
import functools

import jax
import jax.numpy as jnp
from jax import lax
from jax.experimental import pallas as pl
from jax.experimental.pallas import tpu as pltpu
from jax.experimental.pallas import tpu_sc as plsc

D = 32
DP = 128
S = 100
SH = 50
CB = 16
RPC = SH * CB


@functools.lru_cache(maxsize=None)
def _make_gather(BT, V):
    info = plsc.get_sparse_core_info()
    NC, NS, L = info.num_cores, info.num_subcores, info.num_lanes
    NW = NC * NS
    assert L == CB and S % SH == 0
    assert BT % (NW * CB) == 0
    b_per_w = BT // NW
    n_chunks = (b_per_w // CB) * (S // SH)
    s_blocks = S // SH
    mesh = plsc.VectorSubcoreMesh(core_axis_name="c", subcore_axis_name="s")

    @functools.partial(
        pl.kernel,
        mesh=mesh,
        compiler_params=pltpu.CompilerParams(
            use_tc_tiling_on_sc=False, needs_layout_passes=False),
        out_type=jax.ShapeDtypeStruct((S, D // 8, BT // 128, 8, 128),
                                      jnp.float32),
        scratch_types=[
            pltpu.VMEM((2, SH, CB), jnp.int32),
            pltpu.VMEM((2, RPC), jnp.int32),
            pltpu.VMEM((2, RPC, D), jnp.float32),
            pltpu.VMEM((2, SH, D // 8, 8, CB), jnp.float32),
            pltpu.SemaphoreType.DMA,
            pltpu.SemaphoreType.DMA,
            pltpu.SemaphoreType.DMA,
            pltpu.SemaphoreType.DMA,
        ],
    )
    def gather_kernel(idx_hbm, table_hbm, out_hbm, idxb, fidx, rows, outb,
                      sem_g0, sem_g1, sem_o0, sem_o1):
        wid = lax.axis_index("s") * NC + lax.axis_index("c")
        b_base = wid * b_per_w
        iota = lax.iota(jnp.int32, L)
        ti_lo = lax.div(iota, 8)
        ti_hi = ti_lo + 2
        rl = lax.rem(iota, 8)

        def chunk_coords(i):
            s0 = lax.rem(i, s_blocks) * SH
            b0 = b_base + lax.div(i, s_blocks) * CB
            return s0, b0

        def stage_and_fire(i, p):
            s0, b0 = chunk_coords(i)
            pltpu.sync_copy(
                idx_hbm.at[pl.ds(s0, SH), pl.ds(b0, CB)], idxb.at[p])

            def flat_body(s, carry):
                fidx[p, pl.ds(s * CB, CB)] = idxb[p, s, :]
                return carry

            lax.fori_loop(0, SH, flat_body, 0)
            pltpu.async_copy(table_hbm.at[fidx.at[p]],
                             rows.at[p], sem_g0 if p == 0 else sem_g1)

        def wait_gather(p):
            pltpu.make_async_copy(
                table_hbm.at[pl.ds(0, RPC)], rows.at[p],
                sem_g0 if p == 0 else sem_g1).wait()

        def out_slice(s0, tj, cl):
            return out_hbm.at[pl.ds(s0, SH), :, tj, :, pl.ds(cl, CB)]

        def wait_out(p):
            pltpu.make_async_copy(
                outb.at[p], out_slice(0, 0, b_base % 128),
                sem_o0 if p == 0 else sem_o1).wait()

        def transpose_and_fire(i, p):
            def s_body(s, carry):
                s_vec = jnp.full((L,), 0, jnp.int32) + s
                for b in range(CB):
                    row = s * CB + b
                    b_vec = jnp.full((L,), b, jnp.int32)
                    v_lo = rows[p, row, pl.ds(0, L)]
                    v_hi = rows[p, row, pl.ds(L, L)]
                    plsc.store_scatter(
                        outb.at[p], [s_vec, ti_lo, rl, b_vec], v_lo)
                    plsc.store_scatter(
                        outb.at[p], [s_vec, ti_hi, rl, b_vec], v_hi)
                return carry

            lax.fori_loop(0, SH, s_body, 0)
            s0, b0 = chunk_coords(i)
            pltpu.async_copy(
                outb.at[p],
                out_slice(s0, lax.div(b0, 128), lax.rem(b0, 128)),
                sem_o0 if p == 0 else sem_o1)

        stage_and_fire(0, 0)

        def body(i, carry):
            p = lax.rem(i, 2)

            @pl.when(i + 1 < n_chunks)
            def _():
                @pl.when(p == 0)
                def _():
                    stage_and_fire(i + 1, 1)

                @pl.when(p == 1)
                def _():
                    stage_and_fire(i + 1, 0)

            @pl.when(p == 0)
            def _():
                wait_gather(0)

                @pl.when(i >= 2)
                def _():
                    wait_out(0)

                transpose_and_fire(i, 0)

            @pl.when(p == 1)
            def _():
                wait_gather(1)

                @pl.when(i >= 2)
                def _():
                    wait_out(1)

                transpose_and_fire(i, 1)

            return carry

        lax.fori_loop(0, n_chunks, body, 0)
        wait_out(0)
        wait_out(1)

    return gather_kernel


def kernel(input, weight):
    idx_t = jnp.transpose(input).astype(jnp.int32)
    out5 = _make_gather(idx_t.shape[1], weight.shape[0])(idx_t, weight)
    BT = idx_t.shape[1]
    out = jnp.transpose(out5, (2, 4, 0, 1, 3))
    return jnp.reshape(out, (BT, S, D))


# speedup vs baseline: 6.1550x; 1.0661x over previous
import functools

import jax
import jax.numpy as jnp
from jax import lax
from jax.experimental import pallas as pl
from jax.experimental.pallas import tpu as pltpu
from jax.experimental.pallas import tpu_sc as plsc

D = 32
DP = 128
S = 100
SH = 50
CB = 16
RPC = SH * CB


@functools.lru_cache(maxsize=None)
def _make_gather(BT, V):
    info = plsc.get_sparse_core_info()
    NC, NS, L = info.num_cores, info.num_subcores, info.num_lanes
    NW = NC * NS
    assert L == CB and S % SH == 0
    assert BT % (NW * CB) == 0
    b_per_w = BT // NW
    n_chunks = (b_per_w // CB) * (S // SH)
    s_blocks = S // SH
    mesh = plsc.VectorSubcoreMesh(core_axis_name="c", subcore_axis_name="s")

    @functools.partial(
        pl.kernel,
        mesh=mesh,
        compiler_params=pltpu.CompilerParams(
            use_tc_tiling_on_sc=False, needs_layout_passes=False),
        out_type=jax.ShapeDtypeStruct((S, D // 8, BT // 128, 8, 128),
                                      jnp.float32),
        scratch_types=[
            pltpu.VMEM((2, SH, CB), jnp.int32),
            pltpu.VMEM((2, RPC), jnp.int32),
            pltpu.VMEM((2, RPC, D), jnp.float32),
            pltpu.VMEM((2, SH, D // 8, 8, CB), jnp.float32),
            pltpu.SemaphoreType.DMA,
            pltpu.SemaphoreType.DMA,
            pltpu.SemaphoreType.DMA,
            pltpu.SemaphoreType.DMA,
        ],
    )
    def gather_kernel(idx_hbm, table_hbm, out_hbm, idxb, fidx, rows, outb,
                      sem_g0, sem_g1, sem_o0, sem_o1):
        wid = lax.axis_index("s") * NC + lax.axis_index("c")
        b_base = wid * b_per_w
        iota = lax.iota(jnp.int32, L)
        ti_lo = lax.div(iota, 8)
        ti_hi = ti_lo + 2
        rl = lax.rem(iota, 8)

        def chunk_coords(i):
            s0 = lax.rem(i, s_blocks) * SH
            b0 = b_base + lax.div(i, s_blocks) * CB
            return s0, b0

        def stage_and_fire(i, p):
            s0, b0 = chunk_coords(i)
            pltpu.sync_copy(
                idx_hbm.at[pl.ds(s0, SH), pl.ds(b0, CB)], idxb.at[p])

            def flat_body(s, carry):
                fidx[p, pl.ds(s * CB, CB)] = idxb[p, s, :]
                return carry

            lax.fori_loop(0, SH, flat_body, 0)
            pltpu.async_copy(table_hbm.at[fidx.at[p]],
                             rows.at[p], sem_g0 if p == 0 else sem_g1)

        def wait_gather(p):
            pltpu.make_async_copy(
                table_hbm.at[pl.ds(0, RPC)], rows.at[p],
                sem_g0 if p == 0 else sem_g1).wait()

        def out_slice(s0, tj, cl):
            return out_hbm.at[pl.ds(s0, SH), :, tj, :, pl.ds(cl, CB)]

        def wait_out(p):
            pltpu.make_async_copy(
                outb.at[p], out_slice(0, 0, b_base % 128),
                sem_o0 if p == 0 else sem_o1).wait()

        def transpose_and_fire(i, p):
            def s_body(s, carry):
                s_vec = jnp.full((L,), 0, jnp.int32) + s
                for b in range(0, CB, 2):
                    r0 = s * CB + b
                    v = [rows[p, r0, pl.ds(0, L)],
                         rows[p, r0, pl.ds(L, L)],
                         rows[p, r0 + 1, pl.ds(0, L)],
                         rows[p, r0 + 1, pl.ds(L, L)]]
                    b_vec = jnp.full((L,), b, jnp.int32)
                    b_vec1 = jnp.full((L,), b + 1, jnp.int32)
                    plsc.store_scatter(
                        outb.at[p], [s_vec, ti_lo, rl, b_vec], v[0])
                    plsc.store_scatter(
                        outb.at[p], [s_vec, ti_hi, rl, b_vec], v[1])
                    plsc.store_scatter(
                        outb.at[p], [s_vec, ti_lo, rl, b_vec1], v[2])
                    plsc.store_scatter(
                        outb.at[p], [s_vec, ti_hi, rl, b_vec1], v[3])
                return carry

            lax.fori_loop(0, SH, s_body, 0)
            s0, b0 = chunk_coords(i)
            pltpu.async_copy(
                outb.at[p],
                out_slice(s0, lax.div(b0, 128), lax.rem(b0, 128)),
                sem_o0 if p == 0 else sem_o1)

        stage_and_fire(0, 0)

        def body(i, carry):
            p = lax.rem(i, 2)

            @pl.when(i + 1 < n_chunks)
            def _():
                @pl.when(p == 0)
                def _():
                    stage_and_fire(i + 1, 1)

                @pl.when(p == 1)
                def _():
                    stage_and_fire(i + 1, 0)

            @pl.when(p == 0)
            def _():
                wait_gather(0)

                @pl.when(i >= 2)
                def _():
                    wait_out(0)

                transpose_and_fire(i, 0)

            @pl.when(p == 1)
            def _():
                wait_gather(1)

                @pl.when(i >= 2)
                def _():
                    wait_out(1)

                transpose_and_fire(i, 1)

            return carry

        lax.fori_loop(0, n_chunks, body, 0)
        wait_out(0)
        wait_out(1)

    return gather_kernel


def kernel(input, weight):
    idx_t = jnp.transpose(input).astype(jnp.int32)
    out5 = _make_gather(idx_t.shape[1], weight.shape[0])(idx_t, weight)
    BT = idx_t.shape[1]
    out = jnp.transpose(out5, (2, 4, 0, 1, 3))
    return jnp.reshape(out, (BT, S, D))
